# baseline (speedup 1.0000x reference)
import jax
import jax.numpy as jnp
from jax.experimental import pallas as pl

_K = 20
_ROWS = 256
_NPAD = 10240
_NCHUNKS = 8
_CW = _NPAD // _NCHUNKS
_NEG = -3.0
_BIG = 2**30


def _norm_body(emb_ref, out_ref):
    x = emb_ref[...]
    sq = jnp.sum(x * x, axis=1, keepdims=True)
    out_ref[...] = x * jax.lax.rsqrt(jnp.maximum(sq, 1e-12))


def _select20(sim, col, n_valid):
    sim = jnp.where(col < n_valid, sim, _NEG)
    vals = []
    idxs = []
    for _ in range(_K):
        m = jnp.max(sim, axis=1, keepdims=True)
        idx = jnp.min(jnp.where(sim >= m, col, _BIG), axis=1, keepdims=True)
        sim = jnp.where(col == idx, _NEG, sim)
        vals.append(m)
        idxs.append(idx)
    return jnp.concatenate(vals, axis=1), jnp.concatenate(idxs, axis=1)


def _make_topk_body(n_valid):
    def _topk_body(lhs_ref, rhs_ref, vals_ref, idxs_ref):
        i = pl.program_id(0)
        a = lhs_ref[...]
        row = i * _ROWS + jax.lax.broadcasted_iota(jnp.int32, (a.shape[0], 1), 0)

        cand_v = []
        cand_i = []
        for c in range(_NCHUNKS):
            b = rhs_ref[c * _CW:(c + 1) * _CW, :]
            sim = jax.lax.dot_general(
                a, b, (((1,), (1,)), ((), ())),
                preferred_element_type=jnp.float32,
            )
            col = c * _CW + jax.lax.broadcasted_iota(jnp.int32, sim.shape, 1)
            v, ix = _select20(sim, col, n_valid)
            cand_v.append(v)
            cand_i.append(ix)
        cv = jnp.concatenate(cand_v, axis=1)
        ci = jnp.concatenate(cand_i, axis=1)

        vals = []
        idxs = []
        for _ in range(_K):
            m = jnp.max(cv, axis=1, keepdims=True)
            idx = jnp.min(jnp.where(cv >= m, ci, _BIG), axis=1, keepdims=True)
            cv = jnp.where(ci == idx, _NEG, cv)
            vals.append(jnp.where(idx == row, 0.0, m))
            idxs.append(idx)
        vals_ref[...] = jnp.concatenate(vals, axis=1)
        idxs_ref[...] = jnp.concatenate(idxs, axis=1)

    return _topk_body


def kernel(embeddings):
    n, d = embeddings.shape
    norm = pl.pallas_call(
        _norm_body,
        out_shape=jax.ShapeDtypeStruct((n, d), jnp.float32),
    )(embeddings)
    norm = jnp.pad(norm, ((0, _NPAD - n), (0, 0)))

    grid = (pl.cdiv(n, _ROWS),)
    vals, idxs = pl.pallas_call(
        _make_topk_body(n),
        grid=grid,
        in_specs=[
            pl.BlockSpec((_ROWS, d), lambda i: (i, 0)),
            pl.BlockSpec((_NPAD, d), lambda i: (0, 0)),
        ],
        out_specs=[
            pl.BlockSpec((_ROWS, _K), lambda i: (i, 0)),
            pl.BlockSpec((_ROWS, _K), lambda i: (i, 0)),
        ],
        out_shape=[
            jax.ShapeDtypeStruct((n, _K), jnp.float32),
            jax.ShapeDtypeStruct((n, _K), jnp.int32),
        ],
    )(norm, norm)
    return vals, idxs


# speedup vs baseline: 1.5486x; 1.5486x over previous
import jax
import jax.numpy as jnp
from jax.experimental import pallas as pl
from jax.experimental.pallas import tpu as pltpu

_K = 20
_ROWS = 512
_NEG = -3.0
_BIG = float(2**24)


def _norm_body(emb_ref, out_ref):
    x = emb_ref[...]
    sq = jnp.sum(x * x, axis=1, keepdims=True)
    out_ref[...] = x * jax.lax.rsqrt(jnp.maximum(sq, 1e-12))


def _topk_body(lhs_ref, rhs_ref, vals_ref, idxs_ref):
    i = pl.program_id(0)
    a = lhs_ref[...]
    b = rhs_ref[...]
    n = b.shape[0]
    sim = jax.lax.dot_general(
        a, b, (((1,), (1,)), ((), ())), preferred_element_type=jnp.float32
    )
    col = jax.lax.broadcasted_iota(jnp.int32, sim.shape, 1).astype(jnp.float32)
    row = (
        jax.lax.broadcasted_iota(jnp.int32, (sim.shape[0], 1), 0) + i * _ROWS
    ).astype(jnp.float32)
    vals = []
    idxs = []
    for _ in range(_K):
        m = jnp.max(sim, axis=1, keepdims=True)
        idx = jnp.min(jnp.where(sim >= m, col, _BIG), axis=1, keepdims=True)
        sim = jnp.where(col == idx, _NEG, sim)
        vals.append(jnp.where(idx == row, 0.0, m))
        idxs.append(idx)
    vals_ref[...] = jnp.concatenate(vals, axis=1)
    idxs_ref[...] = jnp.concatenate(idxs, axis=1).astype(jnp.int32)


def kernel(embeddings):
    n, d = embeddings.shape
    norm = pl.pallas_call(
        _norm_body,
        out_shape=jax.ShapeDtypeStruct((n, d), jnp.float32),
    )(embeddings)

    grid = (pl.cdiv(n, _ROWS),)
    vals, idxs = pl.pallas_call(
        _topk_body,
        grid=grid,
        in_specs=[
            pl.BlockSpec((_ROWS, d), lambda i: (i, 0)),
            pl.BlockSpec((n, d), lambda i: (0, 0)),
        ],
        out_specs=[
            pl.BlockSpec((_ROWS, _K), lambda i: (i, 0)),
            pl.BlockSpec((_ROWS, _K), lambda i: (i, 0)),
        ],
        out_shape=[
            jax.ShapeDtypeStruct((n, _K), jnp.float32),
            jax.ShapeDtypeStruct((n, _K), jnp.int32),
        ],
        compiler_params=pltpu.CompilerParams(
            dimension_semantics=("parallel",),
        ),
    )(norm, norm)
    return vals, idxs
